# initial kernel scaffold (unmeasured)
import jax
import jax.numpy as jnp
from jax import lax
from jax.experimental import pallas as pl
from jax.experimental.pallas import tpu as pltpu


def kernel(
    x,
):
    def body(*refs):
        pass

    out_shape = jax.ShapeDtypeStruct(..., jnp.float32)
    return pl.pallas_call(body, out_shape=out_shape)(...)


# baseline (device time: 537036 ns/iter reference)
import jax
import jax.numpy as jnp
from jax import lax
from jax.experimental import pallas as pl
from jax.experimental.pallas import tpu as pltpu


def kernel(x):
    m, n2 = x.shape
    n = n2 // 2

    def body(x_ref, out_ref, send_sem, recv_sem, local_sem):
        my_x = lax.axis_index("x")
        my_y = lax.axis_index("y")
        my_z = lax.axis_index("z")
        peer = (my_x, 1 - my_y, my_z)

        barrier = pltpu.get_barrier_semaphore()
        pl.semaphore_signal(
            barrier, inc=1, device_id=peer,
            device_id_type=pl.DeviceIdType.MESH,
        )
        pl.semaphore_wait(barrier, 1)

        def exchange(my_row_off, my_col_off, peer_col_off):
            rdma = pltpu.make_async_remote_copy(
                src_ref=x_ref.at[:, pl.ds(peer_col_off, n)],
                dst_ref=out_ref.at[pl.ds(my_row_off, m), :],
                send_sem=send_sem,
                recv_sem=recv_sem,
                device_id=peer,
                device_id_type=pl.DeviceIdType.MESH,
            )
            rdma.start()
            local = pltpu.make_async_copy(
                x_ref.at[:, pl.ds(my_col_off, n)],
                out_ref.at[pl.ds(my_row_off, m), :],
                local_sem,
            )
            local.start()
            local.wait()
            rdma.wait()

        @pl.when(my_y == 0)
        def _():
            exchange(0, 0, n)

        @pl.when(my_y == 1)
        def _():
            exchange(m, n, 0)

    return pl.pallas_call(
        body,
        out_shape=jax.ShapeDtypeStruct((2 * m, n), x.dtype),
        in_specs=[pl.BlockSpec(memory_space=pl.ANY)],
        out_specs=pl.BlockSpec(memory_space=pl.ANY),
        scratch_shapes=[
            pltpu.SemaphoreType.DMA,
            pltpu.SemaphoreType.DMA,
            pltpu.SemaphoreType.DMA,
        ],
        compiler_params=pltpu.CompilerParams(collective_id=0),
    )(x)


# device time: 106760 ns/iter; 5.0303x vs baseline; 5.0303x over previous
import jax
import jax.numpy as jnp
from jax import lax
from jax.experimental import pallas as pl
from jax.experimental.pallas import tpu as pltpu

N_QUARTERS = 4
CPQ = 8
HPQ = CPQ // 2


def kernel(x):
    m, n2 = x.shape
    n = n2 // 2
    qr = m // N_QUARTERS
    ch = qr // CPQ

    def body(x_ref, out_ref,
             y_snd, y_rcv, xa_snd, xa_rcv, za_snd, za_rcv,
             xb_snd, xb_rcv, zb_snd, zb_rcv,
             local_sems, stage_ref):
        my_x = lax.axis_index("x")
        my_y = lax.axis_index("y")
        my_z = lax.axis_index("z")
        y_peer = (my_x, 1 - my_y, my_z)
        x_peer = (1 - my_x, my_y, my_z)
        z_peer = (my_x, my_y, 1 - my_z)

        j_me = 2 * my_x + my_z
        j_xp = 2 * (1 - my_x) + my_z
        j_zp = 2 * my_x + (1 - my_z)
        j_dg = 2 * (1 - my_x) + (1 - my_z)
        my_row = my_y * m
        p_row = (1 - my_y) * m
        my_col = my_y * n
        peer_col = (1 - my_y) * n

        barrier = pltpu.get_barrier_semaphore()
        for nbr in (y_peer, x_peer, z_peer):
            pl.semaphore_signal(
                barrier, inc=1, device_id=nbr,
                device_id_type=pl.DeviceIdType.MESH,
            )
        pl.semaphore_wait(barrier, 3)

        def rdma(src, dst, ssem, rsem, dev):
            return pltpu.make_async_remote_copy(
                src_ref=src, dst_ref=dst, send_sem=ssem, recv_sem=rsem,
                device_id=dev, device_id_type=pl.DeviceIdType.MESH,
            )

        def prow_chunk(j, c):
            return out_ref.at[pl.ds(p_row + j * qr + c * ch, ch), :]

        y_rdmas = []
        for c in range(CPQ):
            r0 = j_me * qr + c * ch
            rd = rdma(
                x_ref.at[pl.ds(r0, ch), pl.ds(peer_col, n)],
                out_ref.at[pl.ds(my_y * m + r0, ch), :],
                y_snd.at[c], y_rcv.at[c], y_peer,
            )
            rd.start()
            y_rdmas.append(rd)

        cin = pltpu.make_async_copy(
            x_ref.at[:, pl.ds(my_col, n)], stage_ref, local_sems.at[0],
        )
        cin.start()

        xa_rdmas, za_rdmas = [], []
        for c in range(CPQ):
            y_rdmas[c].wait_recv()
            src = prow_chunk(j_me, c)
            ra = rdma(src, prow_chunk(j_me, c), xa_snd.at[c], xa_rcv.at[c],
                      x_peer)
            ra.start()
            xa_rdmas.append(ra)
            rz = rdma(src, prow_chunk(j_me, c), za_snd.at[c], za_rcv.at[c],
                      z_peer)
            rz.start()
            za_rdmas.append(rz)

        cin.wait()
        cout = pltpu.make_async_copy(
            stage_ref, out_ref.at[pl.ds(my_row, m), :], local_sems.at[1],
        )
        cout.start()

        xb_rdmas, zb_rdmas = [], []
        for c in range(HPQ):
            za_rdmas[c].wait_recv()
            rb = rdma(prow_chunk(j_zp, c), prow_chunk(j_zp, c),
                      xb_snd.at[c], xb_rcv.at[c], x_peer)
            rb.start()
            xb_rdmas.append(rb)
        for c in range(HPQ, CPQ):
            xa_rdmas[c].wait_recv()
            rb = rdma(prow_chunk(j_xp, c), prow_chunk(j_xp, c),
                      zb_snd.at[c - HPQ], zb_rcv.at[c - HPQ], z_peer)
            rb.start()
            zb_rdmas.append(rb)

        for c in range(HPQ):
            xa_rdmas[c].wait_recv()
        for c in range(HPQ, CPQ):
            za_rdmas[c].wait_recv()
        for c in range(HPQ):
            rdma(prow_chunk(j_dg, c), prow_chunk(j_dg, c),
                 xb_snd.at[c], xb_rcv.at[c], x_peer).wait_recv()
            rdma(prow_chunk(j_dg, HPQ + c), prow_chunk(j_dg, HPQ + c),
                 zb_snd.at[c], zb_rcv.at[c], z_peer).wait_recv()
        for rd in y_rdmas + xa_rdmas + za_rdmas + xb_rdmas + zb_rdmas:
            rd.wait_send()
        cout.wait()

    sem = pltpu.SemaphoreType.DMA
    return pl.pallas_call(
        body,
        out_shape=jax.ShapeDtypeStruct((2 * m, n), x.dtype),
        in_specs=[pl.BlockSpec(memory_space=pl.ANY)],
        out_specs=pl.BlockSpec(memory_space=pl.ANY),
        scratch_shapes=[
            sem((CPQ,)), sem((CPQ,)),
            sem((CPQ,)), sem((CPQ,)),
            sem((CPQ,)), sem((CPQ,)),
            sem((HPQ,)), sem((HPQ,)),
            sem((HPQ,)), sem((HPQ,)),
            sem((2,)),
            pltpu.VMEM((m, n), x.dtype),
        ],
        compiler_params=pltpu.CompilerParams(collective_id=0),
    )(x)
